# Initial kernel scaffold; baseline (speedup 1.0000x reference)
import jax
import jax.numpy as jnp
from jax.experimental import pallas as pl


def kernel(triples, h_parents, h_children, t_parents, t_children, ent_emb, rel_emb, W_p, W_c, scorer_p_w, scorer_p_b, scorer_c_w, scorer_c_b, gen_w):
    raise NotImplementedError("write your pallas kernel here")


# speedup vs baseline: 2.9011x; 2.9011x over previous
import functools

import jax
import jax.numpy as jnp
from jax import lax
from jax.experimental import pallas as pl
from jax.experimental.pallas import tpu as pltpu
from jax.experimental.pallas import tpu_sc as plsc

DIM = 256
ATTN = 32
L = 16
SLOPE = 0.2
_f32 = jnp.float32

_NC = 2
_NS = 16
_NW = _NC * _NS


def _build_gather(B):
    HT = 2 * B
    NN = 4 * B * L
    NR = B
    ht_pw, n_pw, r_pw = HT // _NW, NN // _NW, NR // _NW
    CH = 128
    n_chunks = n_pw // CH

    mesh = plsc.VectorSubcoreMesh(core_axis_name="c", subcore_axis_name="s")

    @functools.partial(
        pl.kernel,
        mesh=mesh,
        out_type=[
            jax.ShapeDtypeStruct((HT, DIM), _f32),
            jax.ShapeDtypeStruct((NN, DIM), _f32),
            jax.ShapeDtypeStruct((NR, DIM), _f32),
        ],
        scratch_types=[
            pltpu.VMEM((ht_pw,), jnp.int32),
            pltpu.VMEM((ht_pw, DIM), _f32),
            pltpu.VMEM((r_pw,), jnp.int32),
            pltpu.VMEM((r_pw, DIM), _f32),
            pltpu.VMEM((CH,), jnp.int32),
            pltpu.VMEM((CH, DIM), _f32),
            pltpu.SemaphoreType.DMA,
            pltpu.SemaphoreType.DMA,
            pltpu.SemaphoreType.DMA,
        ],
    )
    def gk(ent_hbm, rel_hbm, htidx_hbm, nidx_hbm, ridx_hbm,
           out_ht, out_n, out_r,
           idx_ht, rows_ht, idx_r, rows_r, idx_n, rows_n,
           sem_ht, sem_r, sem_n):
        wid = lax.axis_index("s") * _NC + lax.axis_index("c")
        b_ht = wid * ht_pw
        b_r = wid * r_pw
        b_n = wid * n_pw
        pltpu.sync_copy(htidx_hbm.at[pl.ds(b_ht, ht_pw)], idx_ht)
        cp_ht = pltpu.async_copy(ent_hbm.at[idx_ht], rows_ht, sem_ht)
        pltpu.sync_copy(ridx_hbm.at[pl.ds(b_r, r_pw)], idx_r)
        cp_r = pltpu.async_copy(rel_hbm.at[idx_r], rows_r, sem_r)

        def body(i, carry):
            off = b_n + i * CH
            pltpu.sync_copy(nidx_hbm.at[pl.ds(off, CH)], idx_n)
            pltpu.async_copy(ent_hbm.at[idx_n], rows_n, sem_n).wait()
            pltpu.sync_copy(rows_n, out_n.at[pl.ds(off, CH)])
            return carry

        lax.fori_loop(0, n_chunks, body, 0)
        cp_ht.wait()
        pltpu.sync_copy(rows_ht, out_ht.at[pl.ds(b_ht, ht_pw)])
        cp_r.wait()
        pltpu.sync_copy(rows_r, out_r.at[pl.ds(b_r, r_pw)])

    return gk


def _pre_body(sw_ref, wp_ref, wc_ref, out_ref):
    out_ref[0:2, :] = jnp.dot(sw_ref[0:2, :], wp_ref[...],
                              preferred_element_type=_f32)
    out_ref[2:4, :] = jnp.dot(sw_ref[2:4, :], wc_ref[...],
                              preferred_element_type=_f32)


def _att(es, u, v, n3, bias):
    s_s = jnp.sum(u * es, axis=1, keepdims=True)
    sn = jnp.sum(v[:, None, :] * n3, axis=2)
    sc = s_s + sn + bias
    sc = jnp.where(sc >= 0, sc, SLOPE * sc)
    m = jnp.max(sc, axis=1, keepdims=True)
    e = jnp.exp(sc - m)
    a = e / jnp.sum(e, axis=1, keepdims=True)
    return jnp.sum(a[:, :, None] * n3, axis=1)


def _make_main_body(bB):
    def _main_body(r_ref, h_ref, t_ref, nhp_ref, nhc_ref, ntp_ref, ntc_ref,
                   wsn_ref, gw_ref, pb_ref, cb_ref, out_ref):
        r = r_ref[...]
        eh = h_ref[...]
        et = t_ref[...]
        u_p = jnp.dot(r, wsn_ref[0], preferred_element_type=_f32)
        v_p = jnp.dot(r, wsn_ref[1], preferred_element_type=_f32)
        u_c = jnp.dot(r, wsn_ref[2], preferred_element_type=_f32)
        v_c = jnp.dot(r, wsn_ref[3], preferred_element_type=_f32)
        pb = pb_ref[0, 0]
        cb = cb_ref[0, 0]
        agg_hp = _att(eh, u_p, v_p, nhp_ref[...], pb)
        agg_hc = _att(eh, u_c, v_c, nhc_ref[...], cb)
        agg_tp = _att(et, u_p, v_p, ntp_ref[...], pb)
        agg_tc = _att(et, u_c, v_c, ntc_ref[...], cb)
        cat = jnp.concatenate([
            jnp.concatenate([eh, agg_hp, agg_hc], axis=1),
            jnp.concatenate([et, agg_tp, agg_tc], axis=1)], axis=0)
        gen = jnp.maximum(jnp.dot(cat, gw_ref[...],
                                  preferred_element_type=_f32), 0.0)
        gh = gen[0:bB]
        gt = gen[bB:2 * bB]
        hn = gh / (jnp.sqrt(jnp.sum(gh * gh, axis=1, keepdims=True)) + 1e-12)
        tn = gt / (jnp.sqrt(jnp.sum(gt * gt, axis=1, keepdims=True)) + 1e-12)
        rn = r / (jnp.sqrt(jnp.sum(r * r, axis=1, keepdims=True)) + 1e-12)
        out_ref[...] = jnp.sum(jnp.abs(hn + rn - tn), axis=1, keepdims=True)
    return _main_body


def kernel(triples, h_parents, h_children, t_parents, t_children,
           ent_emb, rel_emb, W_p, W_c,
           scorer_p_w, scorer_p_b, scorer_c_w, scorer_c_b, gen_w):
    B = triples.shape[0]
    i32 = jnp.int32
    h_idx = triples[:, 0].astype(i32)
    r_idx = triples[:, 1].astype(i32)
    t_idx = triples[:, 2].astype(i32)
    ht_idx = jnp.concatenate([h_idx, t_idx], axis=0)
    n_idx = jnp.concatenate([
        h_parents.astype(i32).reshape(-1),
        h_children.astype(i32).reshape(-1),
        t_parents.astype(i32).reshape(-1),
        t_children.astype(i32).reshape(-1)], axis=0)

    ht_rows, n_rows, r_rows = _build_gather(B)(
        ent_emb, rel_emb, ht_idx, n_idx, r_idx)

    sw = jnp.concatenate([scorer_p_w.reshape(2, ATTN),
                          scorer_c_w.reshape(2, ATTN)], axis=0)
    wsn = pl.pallas_call(
        _pre_body,
        out_shape=jax.ShapeDtypeStruct((4, DIM * DIM), _f32),
    )(sw, W_p.reshape(ATTN, DIM * DIM), W_c.reshape(ATTN, DIM * DIM))
    wsn = wsn.reshape(4, DIM, DIM)

    n4 = n_rows.reshape(4 * B, L, DIM)
    gen_wT = gen_w.T
    pb = scorer_p_b.reshape(1, 1)
    cb = scorer_c_b.reshape(1, 1)

    bB = 256
    bpg = B // bB
    score = pl.pallas_call(
        _make_main_body(bB),
        grid=(B // bB,),
        in_specs=[
            pl.BlockSpec((bB, DIM), lambda i: (i, 0)),
            pl.BlockSpec((bB, DIM), lambda i: (i, 0)),
            pl.BlockSpec((bB, DIM), lambda i: (i + bpg, 0)),
            pl.BlockSpec((bB, L, DIM), lambda i: (i, 0, 0)),
            pl.BlockSpec((bB, L, DIM), lambda i: (i + bpg, 0, 0)),
            pl.BlockSpec((bB, L, DIM), lambda i: (i + 2 * bpg, 0, 0)),
            pl.BlockSpec((bB, L, DIM), lambda i: (i + 3 * bpg, 0, 0)),
            pl.BlockSpec((4, DIM, DIM), lambda i: (0, 0, 0)),
            pl.BlockSpec((3 * DIM, DIM), lambda i: (0, 0)),
            pl.BlockSpec(memory_space=pltpu.SMEM),
            pl.BlockSpec(memory_space=pltpu.SMEM),
        ],
        out_specs=pl.BlockSpec((bB, 1), lambda i: (i, 0)),
        out_shape=jax.ShapeDtypeStruct((B, 1), _f32),
    )(r_rows, ht_rows, ht_rows, n4, n4, n4, n4, wsn, gen_wT, pb, cb)
    return score.reshape(B)


# speedup vs baseline: 3.2819x; 1.1313x over previous
import functools

import jax
import jax.numpy as jnp
from jax import lax
from jax.experimental import pallas as pl
from jax.experimental.pallas import tpu as pltpu
from jax.experimental.pallas import tpu_sc as plsc

DIM = 256
ATTN = 32
L = 16
SLOPE = 0.2
_f32 = jnp.float32

_NC = 2
_NS = 16
_NW = _NC * _NS


def _build_gather(B):
    HT = 2 * B
    NN = 4 * B * L
    NR = B
    ht_pw, n_pw, r_pw = HT // _NW, NN // _NW, NR // _NW
    CH = 128
    n_chunks = n_pw // CH

    mesh = plsc.VectorSubcoreMesh(core_axis_name="c", subcore_axis_name="s")

    @functools.partial(
        pl.kernel,
        mesh=mesh,
        out_type=[
            jax.ShapeDtypeStruct((HT, DIM), _f32),
            jax.ShapeDtypeStruct((NN, DIM), _f32),
            jax.ShapeDtypeStruct((NR, DIM), _f32),
        ],
        scratch_types=[
            pltpu.VMEM((ht_pw,), jnp.int32),
            pltpu.VMEM((ht_pw, DIM), _f32),
            pltpu.VMEM((r_pw,), jnp.int32),
            pltpu.VMEM((r_pw, DIM), _f32),
            pltpu.VMEM((CH,), jnp.int32),
            pltpu.VMEM((CH, DIM), _f32),
            pltpu.VMEM((CH,), jnp.int32),
            pltpu.VMEM((CH, DIM), _f32),
            pltpu.SemaphoreType.DMA,
            pltpu.SemaphoreType.DMA,
            pltpu.SemaphoreType.DMA,
            pltpu.SemaphoreType.DMA,
        ],
    )
    def gk(ent_hbm, rel_hbm, htidx_hbm, nidx_hbm, ridx_hbm,
           out_ht, out_n, out_r,
           idx_ht, rows_ht, idx_r, rows_r, idx_n0, rows_n0, idx_n1, rows_n1,
           sem_ht, sem_r, sem_n0, sem_n1):
        wid = lax.axis_index("s") * _NC + lax.axis_index("c")
        b_ht = wid * ht_pw
        b_r = wid * r_pw
        b_n = wid * n_pw
        pltpu.sync_copy(htidx_hbm.at[pl.ds(b_ht, ht_pw)], idx_ht)
        cp_ht = pltpu.async_copy(ent_hbm.at[idx_ht], rows_ht, sem_ht)
        pltpu.sync_copy(ridx_hbm.at[pl.ds(b_r, r_pw)], idx_r)
        cp_r = pltpu.async_copy(rel_hbm.at[idx_r], rows_r, sem_r)

        bufs = ((idx_n0, rows_n0, sem_n0), (idx_n1, rows_n1, sem_n1))
        pltpu.sync_copy(nidx_hbm.at[pl.ds(b_n, CH)], idx_n0)
        pltpu.async_copy(ent_hbm.at[idx_n0], rows_n0, sem_n0)
        pltpu.sync_copy(nidx_hbm.at[pl.ds(b_n + CH, CH)], idx_n1)
        pltpu.async_copy(ent_hbm.at[idx_n1], rows_n1, sem_n1)

        def body(p, carry):
            for bsel in range(2):
                idx_b, rows_b, sem_b = bufs[bsel]
                off = b_n + (2 * p + bsel) * CH
                pltpu.make_async_copy(ent_hbm.at[idx_b], rows_b, sem_b).wait()
                pltpu.sync_copy(rows_b, out_n.at[pl.ds(off, CH)])

                @pl.when(2 * p + bsel + 2 < n_chunks)
                def _():
                    pltpu.sync_copy(nidx_hbm.at[pl.ds(off + 2 * CH, CH)],
                                    idx_b)
                    pltpu.async_copy(ent_hbm.at[idx_b], rows_b, sem_b)
            return carry

        lax.fori_loop(0, n_chunks // 2, body, 0)
        cp_ht.wait()
        pltpu.sync_copy(rows_ht, out_ht.at[pl.ds(b_ht, ht_pw)])
        cp_r.wait()
        pltpu.sync_copy(rows_r, out_r.at[pl.ds(b_r, r_pw)])

    return gk


def _pre_body(sw_ref, wp_ref, wc_ref, out_ref):
    out_ref[0:2, :] = jnp.dot(sw_ref[0:2, :], wp_ref[...],
                              preferred_element_type=_f32)
    out_ref[2:4, :] = jnp.dot(sw_ref[2:4, :], wc_ref[...],
                              preferred_element_type=_f32)


def _att(es, u, v, n3, bias):
    s_s = jnp.sum(u * es, axis=1, keepdims=True)
    sn = jnp.sum(v[:, None, :] * n3, axis=2)
    sc = s_s + sn + bias
    sc = jnp.where(sc >= 0, sc, SLOPE * sc)
    m = jnp.max(sc, axis=1, keepdims=True)
    e = jnp.exp(sc - m)
    a = e / jnp.sum(e, axis=1, keepdims=True)
    return jnp.sum(a[:, :, None] * n3, axis=1)


def _make_main_body(bB):
    def _main_body(r_ref, h_ref, t_ref, nhp_ref, nhc_ref, ntp_ref, ntc_ref,
                   wsn_ref, gw_ref, pb_ref, cb_ref, out_ref):
        r = r_ref[...]
        eh = h_ref[...]
        et = t_ref[...]
        u_p = jnp.dot(r, wsn_ref[0], preferred_element_type=_f32)
        v_p = jnp.dot(r, wsn_ref[1], preferred_element_type=_f32)
        u_c = jnp.dot(r, wsn_ref[2], preferred_element_type=_f32)
        v_c = jnp.dot(r, wsn_ref[3], preferred_element_type=_f32)
        pb = pb_ref[0, 0]
        cb = cb_ref[0, 0]
        agg_hp = _att(eh, u_p, v_p, nhp_ref[...], pb)
        agg_hc = _att(eh, u_c, v_c, nhc_ref[...], cb)
        agg_tp = _att(et, u_p, v_p, ntp_ref[...], pb)
        agg_tc = _att(et, u_c, v_c, ntc_ref[...], cb)
        cat = jnp.concatenate([
            jnp.concatenate([eh, agg_hp, agg_hc], axis=1),
            jnp.concatenate([et, agg_tp, agg_tc], axis=1)], axis=0)
        gen = jnp.maximum(jnp.dot(cat, gw_ref[...],
                                  preferred_element_type=_f32), 0.0)
        gh = gen[0:bB]
        gt = gen[bB:2 * bB]
        hn = gh / (jnp.sqrt(jnp.sum(gh * gh, axis=1, keepdims=True)) + 1e-12)
        tn = gt / (jnp.sqrt(jnp.sum(gt * gt, axis=1, keepdims=True)) + 1e-12)
        rn = r / (jnp.sqrt(jnp.sum(r * r, axis=1, keepdims=True)) + 1e-12)
        out_ref[...] = jnp.sum(jnp.abs(hn + rn - tn), axis=1, keepdims=True)
    return _main_body


def kernel(triples, h_parents, h_children, t_parents, t_children,
           ent_emb, rel_emb, W_p, W_c,
           scorer_p_w, scorer_p_b, scorer_c_w, scorer_c_b, gen_w):
    B = triples.shape[0]
    i32 = jnp.int32
    h_idx = triples[:, 0].astype(i32)
    r_idx = triples[:, 1].astype(i32)
    t_idx = triples[:, 2].astype(i32)
    ht_idx = jnp.concatenate([h_idx, t_idx], axis=0)
    n_idx = jnp.concatenate([
        h_parents.astype(i32).reshape(-1),
        h_children.astype(i32).reshape(-1),
        t_parents.astype(i32).reshape(-1),
        t_children.astype(i32).reshape(-1)], axis=0)

    ht_rows, n_rows, r_rows = _build_gather(B)(
        ent_emb, rel_emb, ht_idx, n_idx, r_idx)

    sw = jnp.concatenate([scorer_p_w.reshape(2, ATTN),
                          scorer_c_w.reshape(2, ATTN)], axis=0)
    wsn = pl.pallas_call(
        _pre_body,
        out_shape=jax.ShapeDtypeStruct((4, DIM * DIM), _f32),
    )(sw, W_p.reshape(ATTN, DIM * DIM), W_c.reshape(ATTN, DIM * DIM))
    wsn = wsn.reshape(4, DIM, DIM)

    n4 = n_rows.reshape(4 * B, L, DIM)
    gen_wT = gen_w.T
    pb = scorer_p_b.reshape(1, 1)
    cb = scorer_c_b.reshape(1, 1)

    bB = 256
    bpg = B // bB
    score = pl.pallas_call(
        _make_main_body(bB),
        grid=(B // bB,),
        in_specs=[
            pl.BlockSpec((bB, DIM), lambda i: (i, 0)),
            pl.BlockSpec((bB, DIM), lambda i: (i, 0)),
            pl.BlockSpec((bB, DIM), lambda i: (i + bpg, 0)),
            pl.BlockSpec((bB, L, DIM), lambda i: (i, 0, 0)),
            pl.BlockSpec((bB, L, DIM), lambda i: (i + bpg, 0, 0)),
            pl.BlockSpec((bB, L, DIM), lambda i: (i + 2 * bpg, 0, 0)),
            pl.BlockSpec((bB, L, DIM), lambda i: (i + 3 * bpg, 0, 0)),
            pl.BlockSpec((4, DIM, DIM), lambda i: (0, 0, 0)),
            pl.BlockSpec((3 * DIM, DIM), lambda i: (0, 0)),
            pl.BlockSpec(memory_space=pltpu.SMEM),
            pl.BlockSpec(memory_space=pltpu.SMEM),
        ],
        out_specs=pl.BlockSpec((bB, 1), lambda i: (i, 0)),
        out_shape=jax.ShapeDtypeStruct((B, 1), _f32),
    )(r_rows, ht_rows, ht_rows, n4, n4, n4, n4, wsn, gen_wT, pb, cb)
    return score.reshape(B)
